# baseline (speedup 1.0000x reference)
import functools

import jax
import jax.numpy as jnp
from jax import lax
from jax.experimental import pallas as pl
from jax.experimental.pallas import tpu as pltpu
from jax.experimental.pallas import tpu_sc as plsc

NBINS = 2048
NCLS = 19
NC, NS, L = 2, 16, 16
NW = NC * NS
NROWS = 2 * NCLS
HSIZE = NROWS * NBINS
UNROLL = 8


def _sc_hist(pred3, lbl2):
    B, C, HW = pred3.shape
    CH = (B * HW) // NW
    per_b = HW // CH
    mesh = plsc.VectorSubcoreMesh(core_axis_name="c", subcore_axis_name="s")

    HCH = CH // 2

    @functools.partial(
        pl.kernel,
        mesh=mesh,
        out_type=jax.ShapeDtypeStruct((NW, NROWS, NBINS), jnp.float32),
        scratch_types=[
            pltpu.VMEM((HSIZE,), jnp.float32),
            pltpu.VMEM((CH,), jnp.int32),
            pltpu.VMEM((HCH,), jnp.float32),
            pltpu.VMEM((HCH,), jnp.float32),
            pltpu.SemaphoreType.DMA,
            pltpu.SemaphoreType.DMA,
            pltpu.SemaphoreType.DMA,
            pltpu.SemaphoreType.DMA,
        ],
        compiler_params=pltpu.CompilerParams(needs_layout_passes=False),
    )
    def hist_kernel(pred_hbm, lbl_hbm, out_hbm, histv, lblv, pva, pvb,
                    lsem, sema, semb, dsem):
        wid = lax.axis_index("s") * NC + lax.axis_index("c")
        b = wid // per_b
        h0 = (wid % per_b) * CH

        zero16 = jnp.zeros((L,), jnp.float32)
        one16 = jnp.ones((L,), jnp.float32)

        def src(c, h):
            return pred_hbm.at[b, c, pl.ds(h0 + h * HCH, HCH)]

        pltpu.async_copy(lbl_hbm.at[b, pl.ds(h0, CH)], lblv, lsem)
        pltpu.async_copy(src(0, 0), pva, sema)

        @functools.partial(plsc.parallel_loop, 0, HSIZE // L, unroll=UNROLL)
        def _zero(i):
            histv[pl.ds(i * L, L)] = zero16

        pltpu.make_async_copy(lbl_hbm.at[b, pl.ds(h0, CH)], lblv,
                              lsem).wait()

        def half(c, h, predv):
            hbase = c * NBINS
            gbase = (NCLS + c) * NBINS
            lo = h * HCH

            @functools.partial(plsc.parallel_loop, 0, HCH // L,
                               unroll=UNROLL)
            def _accum(i):
                off = i * L
                p = predv[pl.ds(off, L)]
                lb = lblv[pl.ds(lo + off, L)]
                bi = jnp.minimum((p * float(NBINS)).astype(jnp.int32),
                                 NBINS - 1)
                base = jnp.where(lb == c, gbase, hbase)
                plsc.addupdate_scatter(histv, [base + bi], one16)

        def cbody(c, _):
            pltpu.make_async_copy(src(c, 0), pva, sema).wait()
            pltpu.async_copy(src(c, 1), pvb, semb)
            half(c, 0, pva)
            pltpu.make_async_copy(src(c, 1), pvb, semb).wait()

            @pl.when(c < C - 1)
            def _():
                pltpu.async_copy(src(c + 1, 0), pva, sema)

            half(c, 1, pvb)
            return 0

        lax.fori_loop(0, C, cbody, 0)

        copies = [
            pltpu.async_copy(histv.at[pl.ds(r * NBINS, NBINS)],
                             out_hbm.at[wid, r], dsem)
            for r in range(NROWS)
        ]
        for cp in copies:
            cp.wait()

    return hist_kernel(pred3, lbl2)


RQ, SQ = 16, NBINS // 16
CR = NCLS * RQ


def _tc_finish(hist3):
    def body(h_ref, out_ref):
        f32 = jnp.float32
        hi = lax.Precision.HIGHEST

        def mm(x, y):
            return lax.dot_general(x, y, (((1,), (0,)), ((), ())),
                                   precision=hi, preferred_element_type=f32)

        s = jnp.sum(h_ref[...], axis=0)
        n2 = s[:CR]
        g2 = s[CR:]
        a = lax.broadcasted_iota(jnp.int32, (SQ, SQ), 0)
        sj = lax.broadcasted_iota(jnp.int32, (SQ, SQ), 1)
        sufx = (a > sj).astype(f32)
        pref = (a + sj <= SQ - 2).astype(f32)
        revp = (a + sj == SQ - 1).astype(f32)
        uu = lax.broadcasted_iota(jnp.int32, (CR, CR), 0)
        vv = lax.broadcasted_iota(jnp.int32, (CR, CR), 1)
        sameb = uu // RQ == vv // RQ
        um = uu % RQ
        vm = vv % RQ
        pflip = jnp.where(sameb & (um + vm == RQ - 1), 1.0, 0.0)
        bsuf = jnp.where(sameb & (vm > um), 1.0, 0.0)
        bpre = jnp.where(sameb & (um + vm <= RQ - 2), 1.0, 0.0)
        blk = jnp.where(sameb, 1.0, 0.0)
        wa = mm(n2, sufx)
        wb = mm(pflip, mm(g2, pref))
        n1 = mm(pflip, mm(g2, revp))
        rtn0 = jnp.sum(n2, axis=1, keepdims=True)
        rtg = jnp.sum(g2, axis=1, keepdims=True)
        above0 = wa + mm(bsuf, rtn0)
        above1 = wb + mm(bpre, rtg)
        gts = mm(blk, rtg)
        i_start = above0 + above1
        u_start = gts + above0
        j_start = i_start / jnp.maximum(u_start, 1.0)
        u_mid = u_start + n2
        j_mid = (i_start + n2) / jnp.maximum(u_mid, 1.0)
        ur = lax.broadcasted_iota(jnp.int32, (CR, SQ), 0)
        sr = lax.broadcasted_iota(jnp.int32, (CR, SQ), 1)
        center = (((ur % RQ) * SQ + sr).astype(f32) + 0.5) * (1.0 / NBINS)
        contrib = center * ((j_mid - j_start) + n1 / jnp.maximum(u_mid, 1.0))
        pres = (gts > 0.0).astype(f32)
        npres = jnp.maximum(jnp.sum(pres) * (1.0 / RQ), 1.0)
        out_ref[...] = (jnp.sum(contrib * pres) / npres).reshape(1, 1)

    return pl.pallas_call(
        body,
        out_shape=jax.ShapeDtypeStruct((1, 1), jnp.float32),
    )(hist3)


def kernel(pred, lbl):
    B, C, H, W = pred.shape
    pred3 = pred.reshape(B, C, H * W)
    lbl2 = lbl.reshape(B, H * W).astype(jnp.int32)
    hist = _sc_hist(pred3, lbl2)
    loss = _tc_finish(hist.reshape(NW, 2 * CR, SQ))
    return loss.reshape(())


# speedup vs baseline: 1.1524x; 1.1524x over previous
import functools

import jax
import jax.numpy as jnp
from jax import lax
from jax.experimental import pallas as pl
from jax.experimental.pallas import tpu as pltpu
from jax.experimental.pallas import tpu_sc as plsc

NBINS = 2048
NCLS = 19
NC, NS, L = 2, 16, 16
NW = NC * NS
NROWS = 2 * NCLS
HSIZE = NROWS * NBINS
UNROLL = 8


def _sc_hist(pred3, lbl2):
    B, C, HW = pred3.shape
    CH = (B * HW) // NW
    per_b = HW // CH
    mesh = plsc.VectorSubcoreMesh(core_axis_name="c", subcore_axis_name="s")

    @functools.partial(
        pl.kernel,
        mesh=mesh,
        out_type=jax.ShapeDtypeStruct((NW, NROWS, NBINS), jnp.float32),
        scratch_types=[
            pltpu.VMEM((HSIZE,), jnp.float32),
            pltpu.VMEM((CH,), jnp.int32),
            pltpu.VMEM((CH,), jnp.float32),
            pltpu.SemaphoreType.DMA,
        ],
        compiler_params=pltpu.CompilerParams(needs_layout_passes=False),
    )
    def hist_kernel(pred_hbm, lbl_hbm, out_hbm, histv, lblv, predv, dsem):
        wid = lax.axis_index("s") * NC + lax.axis_index("c")
        b = wid // per_b
        h0 = (wid % per_b) * CH

        zero16 = jnp.zeros((L,), jnp.float32)
        one16 = jnp.ones((L,), jnp.float32)

        @functools.partial(plsc.parallel_loop, 0, HSIZE // L, unroll=UNROLL)
        def _zero(i):
            histv[pl.ds(i * L, L)] = zero16

        pltpu.sync_copy(lbl_hbm.at[b, pl.ds(h0, CH)], lblv)

        def cbody(c, _):
            pltpu.sync_copy(pred_hbm.at[b, c, pl.ds(h0, CH)], predv)
            hbase = c * NBINS
            gbase = (NCLS + c) * NBINS

            @functools.partial(plsc.parallel_loop, 0, CH // L,
                               unroll=UNROLL)
            def _accum(i):
                off = i * L
                p = predv[pl.ds(off, L)]
                lb = lblv[pl.ds(off, L)]
                bi = jnp.minimum((p * float(NBINS)).astype(jnp.int32),
                                 NBINS - 1)
                base = jnp.where(lb == c, gbase, hbase)
                plsc.addupdate_scatter(histv, [base + bi], one16)

            return 0

        lax.fori_loop(0, C, cbody, 0)

        copies = [
            pltpu.async_copy(histv.at[pl.ds(r * NBINS, NBINS)],
                             out_hbm.at[wid, r], dsem)
            for r in range(NROWS)
        ]
        for cp in copies:
            cp.wait()

    return hist_kernel(pred3, lbl2)


def _tc_finish(hist3):
    def body(h_ref, out_ref):
        s = jnp.sum(h_ref[...], axis=0)
        n0 = s[:NCLS]
        gsum = s[NCLS:]
        row = lax.broadcasted_iota(jnp.int32, (NBINS, NBINS), 0)
        col = lax.broadcasted_iota(jnp.int32, (NBINS, NBINS), 1)
        tri = (row <= col).astype(jnp.float32)
        trir = (row + col >= NBINS - 1).astype(jnp.float32)
        dn = (((1,), (0,)), ((), ()))
        cum0 = lax.dot_general(n0, tri, dn, precision=lax.Precision.HIGHEST,
                               preferred_element_type=jnp.float32)
        cum1 = lax.dot_general(gsum, trir, dn,
                               precision=lax.Precision.HIGHEST,
                               preferred_element_type=jnp.float32)
        n1 = cum1 - jnp.concatenate(
            [jnp.zeros((NCLS, 1), jnp.float32), cum1[:, :NBINS - 1]], axis=1)
        tot0 = jnp.sum(n0, axis=1, keepdims=True)
        tot1 = jnp.sum(gsum, axis=1, keepdims=True)
        gts = tot1
        above0 = tot0 - cum0
        above1 = tot1 - cum1
        i_start = above0 + above1
        u_start = gts + above0
        j_start = i_start / jnp.maximum(u_start, 1.0)
        u_mid = u_start + n0
        j_mid = (i_start + n0) / jnp.maximum(u_mid, 1.0)
        center = (lax.broadcasted_iota(jnp.int32, (NCLS, NBINS), 1)
                  .astype(jnp.float32) + 0.5) * (1.0 / NBINS)
        contrib = center * ((j_mid - j_start) + n1 / jnp.maximum(u_mid, 1.0))
        loss_c = jnp.sum(contrib, axis=1, keepdims=True)
        pres = (gts > 0.0).astype(jnp.float32)
        npres = jnp.maximum(jnp.sum(pres, axis=0, keepdims=True), 1.0)
        num = jnp.sum(loss_c * pres, axis=0, keepdims=True)
        out_ref[...] = num / npres

    return pl.pallas_call(
        body,
        out_shape=jax.ShapeDtypeStruct((1, 1), jnp.float32),
    )(hist3)


def kernel(pred, lbl):
    B, C, H, W = pred.shape
    pred3 = pred.reshape(B, C, H * W)
    lbl2 = lbl.reshape(B, H * W).astype(jnp.int32)
    loss = _tc_finish(_sc_hist(pred3, lbl2))
    return loss.reshape(())
